# baseline (speedup 1.0000x reference)
import jax
import jax.numpy as jnp
from jax import lax
from jax.experimental import pallas as pl
from jax.experimental.pallas import tpu as pltpu, tpu_sc as plsc

N = 10000
K = 4
CIN = 256
COUT = 256
HALF = COUT // 2
NC = 2
NS = 16
CHUNK = 64
NBUF = 4
N_PAD = 10240
ROWS_PER_TILE = N_PAD // NS
SUPER_ROWS = 16
PPS = SUPER_ROWS // NBUF
SUPERS = 40
IDX_TILE = SUPERS * SUPER_ROWS
E_PAD = NS * IDX_TILE * CHUNK
T_QUADS = SUPERS * PPS


def _mm_body(x_ref, w_ref, y_ref):
    acc = lax.dot_general(
        x_ref[...], w_ref[...],
        dimension_numbers=(((1,), (1,)), ((), ())),
        preferred_element_type=jnp.float32,
    )
    y_ref[...] = acc[None]


def _precompute_y(x, w):
    bn = 1000
    grid = (NC, K, N // bn)
    return pl.pallas_call(
        _mm_body,
        grid=grid,
        in_specs=[
            pl.BlockSpec((bn, CIN), lambda h, k, nb: (nb, 0)),
            pl.BlockSpec((HALF, CIN), lambda h, k, nb: (h, k)),
        ],
        out_specs=pl.BlockSpec(
            (1, bn, HALF), lambda h, k, nb: (h, k * (N // bn) + nb, 0)),
        out_shape=jax.ShapeDtypeStruct((NC, K * N, HALF), jnp.float32),
    )(x, w)


def _sc_body(y_hbm, g_hbm, n_hbm, v_hbm, z_hbm, out_hbm,
             rows_0, rows_1, rows_2, rows_3, gbuf, nbuf, vbuf, acc,
             sem_g0, sem_g1, sem_g2, sem_g3,
             sem_s0, sem_s1, sem_s2, sem_s3):
    cid = lax.axis_index("c")
    sid = lax.axis_index("s")

    r0 = sid * ROWS_PER_TILE
    pltpu.sync_copy(z_hbm, acc.at[pl.ds(r0, ROWS_PER_TILE)])
    plsc.subcore_barrier()

    ir0 = sid * IDX_TILE
    rows_list = [rows_0, rows_1, rows_2, rows_3]
    sem_g = [sem_g0, sem_g1, sem_g2, sem_g3]
    sem_s = [sem_s0, sem_s1, sem_s2, sem_s3]

    def _scale_chunk(rows_ref, r):
        def _grp(grp, _):
            vals = vbuf[r, pl.ds(grp * 16, 16)]
            e0 = grp * 16
            for i in range(16):
                vb = lax.broadcast(vals[i], (16,))
                for j in range(HALF // 16):
                    sl = pl.ds(j * 16, 16)
                    rows_ref[e0 + i, sl] = rows_ref[e0 + i, sl] * vb
            return 0
        lax.fori_loop(0, CHUNK // 16, _grp, 0)

    def _quad(t, _):
        @pl.when(t > 0)
        def _():
            for rr, ss in zip(rows_list, sem_s):
                pltpu.make_async_copy(rr, acc.at[nbuf.at[0]], ss).wait()

        @pl.when(t % PPS == 0)
        def _():
            ir = ir0 + (t // PPS) * SUPER_ROWS
            pltpu.sync_copy(g_hbm.at[cid, pl.ds(ir, SUPER_ROWS)], gbuf)
            pltpu.sync_copy(n_hbm.at[pl.ds(ir, SUPER_ROWS)], nbuf)
            pltpu.sync_copy(v_hbm.at[pl.ds(ir, SUPER_ROWS)], vbuf)

        rbase = (t % PPS) * NBUF
        hs = [pltpu.async_copy(y_hbm.at[gbuf.at[rbase + q]], rows_list[q],
                               sem_g[q])
              for q in range(NBUF)]
        for q in range(NBUF):
            hs[q].wait()
            _scale_chunk(rows_list[q], rbase + q)
            pltpu.async_copy(rows_list[q], acc.at[nbuf.at[rbase + q]],
                             sem_s[q], add=True)
        return 0
    lax.fori_loop(0, T_QUADS, _quad, 0)

    for rr, ss in zip(rows_list, sem_s):
        pltpu.make_async_copy(rr, acc.at[nbuf.at[0]], ss).wait()
    plsc.subcore_barrier()
    pltpu.sync_copy(acc.at[pl.ds(r0, ROWS_PER_TILE)],
                    out_hbm.at[cid, pl.ds(r0, ROWS_PER_TILE)])


def _sc_aggregate(y_flat, g2, n2, v2):
    mesh = plsc.VectorSubcoreMesh(
        core_axis_name="c", subcore_axis_name="s",
        num_cores=NC, num_subcores=NS)
    call = pl.kernel(
        _sc_body,
        out_type=jax.ShapeDtypeStruct((NC, N_PAD, HALF), jnp.float32),
        mesh=mesh,
        scratch_types=(
            [pltpu.VMEM((CHUNK, HALF), jnp.float32)] * NBUF
            + [pltpu.VMEM((SUPER_ROWS, CHUNK), jnp.int32),
               pltpu.VMEM((SUPER_ROWS, CHUNK), jnp.int32),
               pltpu.VMEM((SUPER_ROWS, CHUNK), jnp.float32),
               pltpu.VMEM_SHARED((N_PAD, HALF), jnp.float32)]
            + [pltpu.SemaphoreType.DMA] * (2 * NBUF)
        ),
    )
    zeros = jnp.zeros((ROWS_PER_TILE, HALF), jnp.float32)
    return call(y_flat, g2, n2, v2, zeros)


def _relu_body(acc_ref, b_ref, o_ref):
    a = jnp.concatenate([acc_ref[0], acc_ref[1]], axis=-1)
    o_ref[...] = jnp.maximum(a + b_ref[...], 0.0)


def _bias_relu(acc2, b):
    bn = 1000
    return pl.pallas_call(
        _relu_body,
        grid=(N // bn,),
        in_specs=[
            pl.BlockSpec((NC, bn, HALF), lambda i: (0, i, 0)),
            pl.BlockSpec((1, COUT), lambda i: (0, 0)),
        ],
        out_specs=pl.BlockSpec((bn, COUT), lambda i: (i, 0)),
        out_shape=jax.ShapeDtypeStruct((N, COUT), jnp.float32),
    )(acc2, b.reshape(1, COUT))


def kernel(x, adj_row, adj_col, adj_val, W, b):
    e = adj_row.shape[0]
    pad = E_PAD - e

    g = (adj_row % K) * N + adj_col
    n_idx = adj_row // K
    g = jnp.pad(g, (0, pad))
    n_idx = jnp.pad(n_idx, (0, pad))
    val = jnp.pad(adj_val, (0, pad))

    nrows = E_PAD // CHUNK
    g2 = jnp.stack([g, g + K * N]).reshape(NC, nrows, CHUNK)
    n2 = n_idx.reshape(nrows, CHUNK)
    v2 = val.reshape(nrows, CHUNK)

    y = _precompute_y(x, W)
    y_flat = y.reshape(NC * K * N, HALF)
    acc2 = _sc_aggregate(y_flat, g2, n2, v2)
    return _bias_relu(acc2, b)


# speedup vs baseline: 1.1599x; 1.1599x over previous
import jax
import jax.numpy as jnp
from jax import lax
from jax.experimental import pallas as pl
from jax.experimental.pallas import tpu as pltpu, tpu_sc as plsc

N = 10000
K = 4
CIN = 256
COUT = 256
HALF = COUT // 2
NC = 2
NS = 16
CHUNK = 64
N_PAD = 10240
ROWS_PER_TILE = N_PAD // NS
SUPER_ROWS = 16
PPS = SUPER_ROWS // 2
SUPERS = 40
IDX_TILE = SUPERS * SUPER_ROWS
E_PAD = NS * IDX_TILE * CHUNK
T_PAIRS = SUPERS * PPS


def _mm_body(x_ref, w_ref, y_ref):
    acc = lax.dot_general(
        x_ref[...], w_ref[...],
        dimension_numbers=(((1,), (1,)), ((), ())),
        preferred_element_type=jnp.float32,
    )
    y_ref[...] = acc[None]


def _precompute_y(x, w):
    bn = 1000
    grid = (NC, K, N // bn)
    return pl.pallas_call(
        _mm_body,
        grid=grid,
        in_specs=[
            pl.BlockSpec((bn, CIN), lambda h, k, nb: (nb, 0)),
            pl.BlockSpec((HALF, CIN), lambda h, k, nb: (h, k)),
        ],
        out_specs=pl.BlockSpec(
            (1, bn, HALF), lambda h, k, nb: (h, k * (N // bn) + nb, 0)),
        out_shape=jax.ShapeDtypeStruct((NC, K * N, HALF), jnp.float32),
    )(x, w)


def _sc_body(y_hbm, g_hbm, n_hbm, v_hbm, z_hbm, out_hbm,
             raw_a, raw_b, scl_a, scl_b, gbuf, nbuf, vbuf, acc,
             sem_ga, sem_gb, sem_sa, sem_sb):
    cid = lax.axis_index("c")
    sid = lax.axis_index("s")

    r0 = sid * ROWS_PER_TILE
    pltpu.sync_copy(z_hbm, acc.at[pl.ds(r0, ROWS_PER_TILE)])
    plsc.subcore_barrier()

    is0 = sid * SUPERS

    def _scale_chunk(raw_ref, scl_ref, h, r):
        def _grp(grp, _):
            vals = vbuf[h, r, pl.ds(grp * 16, 16)]
            e0 = grp * 16
            for i in range(16):
                vb = lax.broadcast(vals[i], (16,))
                for j in range(HALF // 16):
                    sl = pl.ds(j * 16, 16)
                    scl_ref[e0 + i, sl] = raw_ref[e0 + i, sl] * vb
            return 0
        lax.fori_loop(0, CHUNK // 16, _grp, 0)

    pltpu.sync_copy(g_hbm.at[cid, is0], gbuf.at[0])
    pltpu.sync_copy(n_hbm.at[is0], nbuf.at[0])
    pltpu.sync_copy(v_hbm.at[is0], vbuf.at[0])
    pltpu.async_copy(y_hbm.at[gbuf.at[0, 0]], raw_a, sem_ga)
    pltpu.async_copy(y_hbm.at[gbuf.at[0, 1]], raw_b, sem_gb)

    def _pair(t, _):
        s = t // PPS
        boundary = t % PPS == 0

        @pl.when(boundary & (t > 0))
        def _():
            pltpu.make_async_copy(scl_a, acc.at[nbuf.at[0, 0]], sem_sa).wait()
            pltpu.make_async_copy(scl_b, acc.at[nbuf.at[0, 0]], sem_sb).wait()

        @pl.when(boundary & (s + 1 < SUPERS))
        def _():
            half = (s + 1) % 2
            gs = is0 + s + 1
            pltpu.sync_copy(g_hbm.at[cid, gs], gbuf.at[half])
            pltpu.sync_copy(n_hbm.at[gs], nbuf.at[half])
            pltpu.sync_copy(v_hbm.at[gs], vbuf.at[half])

        h = s % 2
        ra = (t % PPS) * 2
        rb = ra + 1
        h2 = ((t + 1) // PPS) % 2
        ra2 = ((t + 1) % PPS) * 2

        pltpu.make_async_copy(y_hbm.at[gbuf.at[h, ra]], raw_a, sem_ga).wait()

        @pl.when((t > 0) & jnp.logical_not(boundary))
        def _():
            pltpu.make_async_copy(scl_a, acc.at[nbuf.at[0, 0]], sem_sa).wait()

        _scale_chunk(raw_a, scl_a, h, ra)

        @pl.when(t + 1 < T_PAIRS)
        def _():
            pltpu.async_copy(y_hbm.at[gbuf.at[h2, ra2]], raw_a, sem_ga)

        pltpu.async_copy(scl_a, acc.at[nbuf.at[h, ra]], sem_sa, add=True)

        pltpu.make_async_copy(y_hbm.at[gbuf.at[h, rb]], raw_b, sem_gb).wait()

        @pl.when((t > 0) & jnp.logical_not(boundary))
        def _():
            pltpu.make_async_copy(scl_b, acc.at[nbuf.at[0, 0]], sem_sb).wait()

        _scale_chunk(raw_b, scl_b, h, rb)

        @pl.when(t + 1 < T_PAIRS)
        def _():
            pltpu.async_copy(y_hbm.at[gbuf.at[h2, ra2 + 1]], raw_b, sem_gb)

        pltpu.async_copy(scl_b, acc.at[nbuf.at[h, rb]], sem_sb, add=True)
        return 0
    lax.fori_loop(0, T_PAIRS, _pair, 0)

    pltpu.make_async_copy(scl_a, acc.at[nbuf.at[0, 0]], sem_sa).wait()
    pltpu.make_async_copy(scl_b, acc.at[nbuf.at[0, 0]], sem_sb).wait()
    plsc.subcore_barrier()
    pltpu.sync_copy(acc.at[pl.ds(r0, ROWS_PER_TILE)],
                    out_hbm.at[cid, pl.ds(r0, ROWS_PER_TILE)])


def _sc_aggregate(y_flat, g2, n2, v2):
    mesh = plsc.VectorSubcoreMesh(
        core_axis_name="c", subcore_axis_name="s",
        num_cores=NC, num_subcores=NS)
    call = pl.kernel(
        _sc_body,
        out_type=jax.ShapeDtypeStruct((NC, N_PAD, HALF), jnp.float32),
        mesh=mesh,
        scratch_types=(
            [pltpu.VMEM((CHUNK, HALF), jnp.float32)] * 4
            + [pltpu.VMEM((2, SUPER_ROWS, CHUNK), jnp.int32),
               pltpu.VMEM((2, SUPER_ROWS, CHUNK), jnp.int32),
               pltpu.VMEM((2, SUPER_ROWS, CHUNK), jnp.float32),
               pltpu.VMEM_SHARED((N_PAD, HALF), jnp.float32)]
            + [pltpu.SemaphoreType.DMA] * 4
        ),
    )
    zeros = jnp.zeros((ROWS_PER_TILE, HALF), jnp.float32)
    return call(y_flat, g2, n2, v2, zeros)


def _relu_body(acc_ref, b_ref, o_ref):
    a = jnp.concatenate([acc_ref[0], acc_ref[1]], axis=-1)
    o_ref[...] = jnp.maximum(a + b_ref[...], 0.0)


def _bias_relu(acc2, b):
    bn = 1000
    return pl.pallas_call(
        _relu_body,
        grid=(N // bn,),
        in_specs=[
            pl.BlockSpec((NC, bn, HALF), lambda i: (0, i, 0)),
            pl.BlockSpec((1, COUT), lambda i: (0, 0)),
        ],
        out_specs=pl.BlockSpec((bn, COUT), lambda i: (i, 0)),
        out_shape=jax.ShapeDtypeStruct((N, COUT), jnp.float32),
    )(acc2, b.reshape(1, COUT))


def kernel(x, adj_row, adj_col, adj_val, W, b):
    e = adj_row.shape[0]
    pad = E_PAD - e

    g = (adj_row % K) * N + adj_col
    n_idx = adj_row // K
    g = jnp.pad(g, (0, pad))
    n_idx = jnp.pad(n_idx, (0, pad))
    val = jnp.pad(adj_val, (0, pad))

    nsup = E_PAD // (CHUNK * SUPER_ROWS)
    g2 = jnp.stack([g, g + K * N]).reshape(NC, nsup, SUPER_ROWS, CHUNK)
    n2 = n_idx.reshape(nsup, SUPER_ROWS, CHUNK)
    v2 = val.reshape(nsup, SUPER_ROWS, CHUNK)

    y = _precompute_y(x, W)
    y_flat = y.reshape(NC * K * N, HALF)
    acc2 = _sc_aggregate(y_flat, g2, n2, v2)
    return _bias_relu(acc2, b)
